# baseline (speedup 1.0000x reference)
import functools

import jax
import jax.numpy as jnp
from jax import lax
from jax.experimental import pallas as pl
from jax.experimental.pallas import tpu as pltpu
from jax.experimental.pallas import tpu_sc as plsc

D = 128
S = 10
NC = 2
NS = 16
NW = NC * NS
B_PAD = 52224
R = B_PAD // NW
C = 48
GR = 120
NG = C * S // GR
NSLOT = GR // S
NCHUNK = R // C


def _sc_gather_pool(nodes_hbm, neigh_hbm, feat_hbm, self_out, neigh_out,
                    sidx_a, sidx_b, nidx_a, nidx_b, sbuf, nbuf0, nbuf1, acc,
                    sem_n, sem_s, sem_o, sem_i):
    wid = lax.axis_index("s") * NC + lax.axis_index("c")
    base = wid * R
    nbufs = (nbuf0, nbuf1)

    def stage_idx(j, sidx, nidx):
        cb = base + jnp.minimum(j, NCHUNK - 1) * C
        c1 = pltpu.async_copy(nodes_hbm.at[pl.ds(cb, C)], sidx, sem_i)
        c2 = pltpu.async_copy(neigh_hbm.at[pl.ds(cb * S, C * S)], nidx, sem_i)
        return c1, c2

    def run_chunk(j, sidx, nidx, pending_idx):
        cb = base + j * C
        for cp in pending_idx:
            cp.wait()
        cp_s = pltpu.async_copy(feat_hbm.at[sidx], sbuf, sem_s)
        gathers = [pltpu.async_copy(
            feat_hbm.at[nidx.at[pl.ds(0, GR)]], nbufs[0], sem_n)]
        for g in range(NG):
            gathers[g].wait()
            if g + 1 < NG:
                gathers.append(pltpu.async_copy(
                    feat_hbm.at[nidx.at[pl.ds(GR * (g + 1), GR)]],
                    nbufs[(g + 1) % 2], sem_n))
            nb = nbufs[g % 2]

            def slot_body(c, c2, _g=g, _nb=nb):
                r0 = c * S
                for v in range(8):
                    col = pl.ds(v * 16, 16)
                    a = _nb[r0, col]
                    for s in range(1, S):
                        a = a + _nb[r0 + s, col]
                    acc[_g * NSLOT + c, col] = a
                return c2

            lax.fori_loop(0, NSLOT, slot_body, 0)

        cp_s.wait()
        o1 = pltpu.async_copy(sbuf, self_out.at[pl.ds(cb, C)], sem_o)
        o2 = pltpu.async_copy(acc, neigh_out.at[pl.ds(cb, C)], sem_o)
        o1.wait()
        o2.wait()

    def idx_waits(sidx, nidx):
        return (pltpu.make_async_copy(nodes_hbm.at[pl.ds(0, C)], sidx, sem_i),
                pltpu.make_async_copy(neigh_hbm.at[pl.ds(0, C * S)], nidx, sem_i))

    stage_idx(0, sidx_a, nidx_a)

    def pair_body(t, carry):
        stage_idx(2 * t + 1, sidx_b, nidx_b)
        run_chunk(2 * t, sidx_a, nidx_a, idx_waits(sidx_a, nidx_a))
        stage_idx(2 * t + 2, sidx_a, nidx_a)
        run_chunk(2 * t + 1, sidx_b, nidx_b, idx_waits(sidx_b, nidx_b))
        return carry

    lax.fori_loop(0, NCHUNK // 2, pair_body, 0)

    for cp in idx_waits(sidx_a, nidx_a):
        cp.wait()


def _sc_gather(nodes_p, neigh_p, feat_table):
    mesh = plsc.VectorSubcoreMesh(core_axis_name="c", subcore_axis_name="s")
    fn = functools.partial(
        pl.kernel,
        mesh=mesh,
        out_type=[
            jax.ShapeDtypeStruct((B_PAD, D), jnp.float32),
            jax.ShapeDtypeStruct((B_PAD, D), jnp.float32),
        ],
        scratch_types=[
            pltpu.VMEM((C,), jnp.int32),
            pltpu.VMEM((C,), jnp.int32),
            pltpu.VMEM((C * S,), jnp.int32),
            pltpu.VMEM((C * S,), jnp.int32),
            pltpu.VMEM((C, D), jnp.float32),
            pltpu.VMEM((GR, D), jnp.float32),
            pltpu.VMEM((GR, D), jnp.float32),
            pltpu.VMEM((C, D), jnp.float32),
            pltpu.SemaphoreType.DMA,
            pltpu.SemaphoreType.DMA,
            pltpu.SemaphoreType.DMA,
            pltpu.SemaphoreType.DMA,
        ],
    )(_sc_gather_pool)
    return fn(nodes_p, neigh_p, feat_table)


def _stats_body(x_ref, o_ref):
    @pl.when(pl.program_id(0) == 0)
    def _():
        o_ref[...] = jnp.zeros_like(o_ref)

    x = x_ref[...]
    o_ref[...] += jnp.concatenate(
        [jnp.sum(x, 0)[None, :], jnp.sum(x * x, 0)[None, :]], axis=0)


def _mm_body(nbatch, self_ref, neigh_ref, w_ref, p_ref, o_ref):
    p = p_ref[...]
    mu = p[0] / nbatch
    var = p[1] / nbatch - mu * mu
    scale = p[2] * lax.rsqrt(var + 1e-5)
    bias = p[3] - mu * scale
    s_norm = self_ref[...] * scale[None, :] + bias[None, :]
    w = w_ref[...]
    o = lax.dot_general(w[:, :D], s_norm, (((1,), (1,)), ((), ())),
                        precision=lax.Precision.HIGHEST)
    o = o + lax.dot_general(w[:, D:] * (1.0 / S), neigh_ref[...],
                            (((1,), (1,)), ((), ())),
                            precision=lax.Precision.HIGHEST)
    o_ref[...] = jnp.maximum(o, 0.0)


def kernel(nodes, neigh_idx, feat_table, W, gamma, beta):
    B = nodes.shape[0]
    E = W.shape[0]
    nodes_p = jnp.pad(nodes.astype(jnp.int32), (0, B_PAD - B))
    neigh_p = jnp.pad(neigh_idx.astype(jnp.int32).reshape(-1),
                      (0, (B_PAD - B) * S))
    feat_table = feat_table.astype(jnp.float32)

    self_raw, neigh_sum = _sc_gather(nodes_p, neigh_p, feat_table)

    rows_blk = 1000
    assert B % rows_blk == 0
    stats = pl.pallas_call(
        _stats_body,
        grid=(B // rows_blk,),
        in_specs=[pl.BlockSpec((rows_blk, D), lambda i: (i, 0))],
        out_specs=pl.BlockSpec((2, D), lambda i: (0, 0)),
        out_shape=jax.ShapeDtypeStruct((2, D), jnp.float32),
    )(self_raw)

    params = jnp.concatenate(
        [stats, gamma[None, :].astype(jnp.float32),
         beta[None, :].astype(jnp.float32)], axis=0)

    bn = 512
    out = pl.pallas_call(
        functools.partial(_mm_body, float(B)),
        grid=(B_PAD // bn,),
        in_specs=[
            pl.BlockSpec((bn, D), lambda i: (i, 0)),
            pl.BlockSpec((bn, D), lambda i: (i, 0)),
            pl.BlockSpec((E, 2 * D), lambda i: (0, 0)),
            pl.BlockSpec((4, D), lambda i: (0, 0)),
        ],
        out_specs=pl.BlockSpec((E, bn), lambda i: (0, i)),
        out_shape=jax.ShapeDtypeStruct((E, B_PAD), jnp.float32),
    )(self_raw, neigh_sum, W.astype(jnp.float32), params)

    return out[:, :B]


# speedup vs baseline: 1.3562x; 1.3562x over previous
import functools

import jax
import jax.numpy as jnp
from jax import lax
from jax.experimental import pallas as pl
from jax.experimental.pallas import tpu as pltpu
from jax.experimental.pallas import tpu_sc as plsc

D = 128
S = 10
NC = 2
NS = 16
NW = NC * NS
B_PAD = 51200
R = B_PAD // NW
C = 64
GR = 80
NWIN = C * S // GR
NSLOT = GR // S
NCHUNK = R // C


def _sc_gather_pool(nodes_hbm, neigh_hbm, feat_hbm, self_out, neigh_out,
                    sidx_a, sidx_b, nidx_a, nidx_b, sbuf_a, sbuf_b,
                    acc_a, acc_b, w0, w1, w2, w3, w4, w5, w6, w7,
                    sem_n, sem_s, sem_o, sem_i):
    wid = lax.axis_index("s") * NC + lax.axis_index("c")
    base = wid * R
    wins = (w0, w1, w2, w3, w4, w5, w6, w7)

    def stage_idx(j, sidx, nidx):
        cb = base + jnp.minimum(j, NCHUNK - 1) * C
        pltpu.async_copy(nodes_hbm.at[pl.ds(cb, C)], sidx, sem_i)
        pltpu.async_copy(neigh_hbm.at[pl.ds(cb * S, C * S)], nidx, sem_i)

    def idx_waits(sidx, nidx):
        return (pltpu.make_async_copy(nodes_hbm.at[pl.ds(0, C)], sidx, sem_i),
                pltpu.make_async_copy(neigh_hbm.at[pl.ds(0, C * S)], nidx, sem_i))

    def out_waits(sbuf, acc):
        return (pltpu.make_async_copy(sbuf, self_out.at[pl.ds(0, C)], sem_o),
                pltpu.make_async_copy(acc, neigh_out.at[pl.ds(0, C)], sem_o))

    def run_chunk(j, sidx, nidx, sbuf, acc, nxt, drain_prev):
        cb = base + j * C
        if drain_prev is not None:
            @pl.when(drain_prev)
            def _():
                for cp in out_waits(sbuf, acc):
                    cp.wait()
        for cp in idx_waits(sidx, nidx):
            cp.wait()
        gathers = [pltpu.async_copy(
            feat_hbm.at[nidx.at[pl.ds(GR * w, GR)]], wins[w], sem_n)
            for w in range(NWIN)]
        cp_s = pltpu.async_copy(feat_hbm.at[sidx], sbuf, sem_s)
        if nxt is not None:
            stage_idx(nxt[0], nxt[1], nxt[2])
        for w in range(NWIN):
            gathers[w].wait()
            nb = wins[w]

            def slot_body(c, c2, _w=w, _nb=nb):
                r0 = c * S
                for v in range(8):
                    col = pl.ds(v * 16, 16)
                    a = _nb[r0, col]
                    for s in range(1, S):
                        a = a + _nb[r0 + s, col]
                    acc[_w * NSLOT + c, col] = a
                return c2

            lax.fori_loop(0, NSLOT, slot_body, 0)

        cp_s.wait()
        pltpu.async_copy(sbuf, self_out.at[pl.ds(cb, C)], sem_o)
        pltpu.async_copy(acc, neigh_out.at[pl.ds(cb, C)], sem_o)

    stage_idx(0, sidx_a, nidx_a)

    def pair_body(t, carry):
        run_chunk(2 * t, sidx_a, nidx_a, sbuf_a, acc_a,
                  (2 * t + 1, sidx_b, nidx_b), t > 0)
        run_chunk(2 * t + 1, sidx_b, nidx_b, sbuf_b, acc_b,
                  (2 * t + 2, sidx_a, nidx_a), t > 0)
        return carry

    lax.fori_loop(0, NCHUNK // 2, pair_body, 0)

    for cp in out_waits(sbuf_a, acc_a):
        cp.wait()
    run_chunk(NCHUNK - 1, sidx_a, nidx_a, sbuf_a, acc_a, None, None)

    for cp in out_waits(sbuf_b, acc_b):
        cp.wait()
    for cp in out_waits(sbuf_a, acc_a):
        cp.wait()


def _sc_gather(nodes_p, neigh_p, feat_table):
    mesh = plsc.VectorSubcoreMesh(core_axis_name="c", subcore_axis_name="s")
    fn = functools.partial(
        pl.kernel,
        mesh=mesh,
        out_type=[
            jax.ShapeDtypeStruct((B_PAD, D), jnp.float32),
            jax.ShapeDtypeStruct((B_PAD, D), jnp.float32),
        ],
        scratch_types=[
            pltpu.VMEM((C,), jnp.int32),
            pltpu.VMEM((C,), jnp.int32),
            pltpu.VMEM((C * S,), jnp.int32),
            pltpu.VMEM((C * S,), jnp.int32),
            pltpu.VMEM((C, D), jnp.float32),
            pltpu.VMEM((C, D), jnp.float32),
            pltpu.VMEM((C, D), jnp.float32),
            pltpu.VMEM((C, D), jnp.float32),
        ] + [pltpu.VMEM((GR, D), jnp.float32)] * NWIN + [
            pltpu.SemaphoreType.DMA,
            pltpu.SemaphoreType.DMA,
            pltpu.SemaphoreType.DMA,
            pltpu.SemaphoreType.DMA,
        ],
    )(_sc_gather_pool)
    return fn(nodes_p, neigh_p, feat_table)


def _stats_body(x_ref, o_ref):
    @pl.when(pl.program_id(0) == 0)
    def _():
        o_ref[...] = jnp.zeros_like(o_ref)

    x = x_ref[...]
    o_ref[...] += jnp.concatenate(
        [jnp.sum(x, 0)[None, :], jnp.sum(x * x, 0)[None, :]], axis=0)


def _mm_body(nbatch, self_ref, neigh_ref, w_ref, p_ref, o_ref):
    p = p_ref[...]
    mu = p[0] / nbatch
    var = p[1] / nbatch - mu * mu
    scale = p[2] * lax.rsqrt(var + 1e-5)
    bias = p[3] - mu * scale
    s_norm = self_ref[...] * scale[None, :] + bias[None, :]
    w = w_ref[...]
    o = lax.dot_general(w[:, :D], s_norm, (((1,), (1,)), ((), ())),
                        precision=lax.Precision.HIGHEST)
    o = o + lax.dot_general(w[:, D:] * (1.0 / S), neigh_ref[...],
                            (((1,), (1,)), ((), ())),
                            precision=lax.Precision.HIGHEST)
    o_ref[...] = jnp.maximum(o, 0.0)


def kernel(nodes, neigh_idx, feat_table, W, gamma, beta):
    B = nodes.shape[0]
    E = W.shape[0]
    nodes_p = jnp.pad(nodes.astype(jnp.int32), (0, B_PAD - B))
    neigh_p = jnp.pad(neigh_idx.astype(jnp.int32).reshape(-1),
                      (0, (B_PAD - B) * S))
    feat_table = feat_table.astype(jnp.float32)

    self_raw, neigh_sum = _sc_gather(nodes_p, neigh_p, feat_table)

    rows_blk = 1000
    assert B % rows_blk == 0
    stats = pl.pallas_call(
        _stats_body,
        grid=(B // rows_blk,),
        in_specs=[pl.BlockSpec((rows_blk, D), lambda i: (i, 0))],
        out_specs=pl.BlockSpec((2, D), lambda i: (0, 0)),
        out_shape=jax.ShapeDtypeStruct((2, D), jnp.float32),
    )(self_raw)

    params = jnp.concatenate(
        [stats, gamma[None, :].astype(jnp.float32),
         beta[None, :].astype(jnp.float32)], axis=0)

    bn = 512
    out = pl.pallas_call(
        functools.partial(_mm_body, float(B)),
        grid=(B_PAD // bn,),
        in_specs=[
            pl.BlockSpec((bn, D), lambda i: (i, 0)),
            pl.BlockSpec((bn, D), lambda i: (i, 0)),
            pl.BlockSpec((E, 2 * D), lambda i: (0, 0)),
            pl.BlockSpec((4, D), lambda i: (0, 0)),
        ],
        out_specs=pl.BlockSpec((E, bn), lambda i: (0, i)),
        out_shape=jax.ShapeDtypeStruct((E, B_PAD), jnp.float32),
    )(self_raw, neigh_sum, W.astype(jnp.float32), params)

    return out[:, :B]
